# baseline (speedup 1.0000x reference)
import functools

import jax
import jax.numpy as jnp
from jax import lax
from jax.experimental import pallas as pl
from jax.experimental.pallas import tpu as pltpu
from jax.experimental.pallas import tpu_sc as plsc

B = 16384
D = 128
NC = 2
NS = 16
BLK = 1024

_mesh = plsc.VectorSubcoreMesh(core_axis_name="c", subcore_axis_name="s")


@functools.partial(
    pl.kernel,
    mesh=_mesh,
    out_type=jax.ShapeDtypeStruct((2, D), jnp.float32),
    compiler_params=pltpu.CompilerParams(use_tc_tiling_on_sc=True),
    scratch_types=[
        pltpu.VMEM((1,), jnp.int32),
        pltpu.VMEM((1, D), jnp.float32),
        pltpu.VMEM((1, D), jnp.float32),
        pltpu.SemaphoreType.DMA,
    ],
)
def _lookup_sc(lab_hbm, scale_hbm, off_hbm, rows_hbm, idx_v, srow_v, orow_v, sem):
    wid = lax.axis_index("s") * NC + lax.axis_index("c")

    @pl.when(wid == 0)
    def _():
        pltpu.sync_copy(lab_hbm, idx_v)
        pltpu.async_copy(scale_hbm.at[idx_v], srow_v, sem).wait()
        pltpu.async_copy(off_hbm.at[idx_v], orow_v, sem).wait()
        pltpu.sync_copy(srow_v, rows_hbm.at[pl.ds(0, 1)])
        pltpu.sync_copy(orow_v, rows_hbm.at[pl.ds(1, 1)])


def _affine_tc_body(rows_ref, x_ref, o_ref):
    s = rows_ref[0:1, :]
    o = rows_ref[1:2, :]
    o_ref[...] = jnp.maximum(x_ref[...] * s + o, 0.0)


_affine_tc = pl.pallas_call(
    _affine_tc_body,
    out_shape=jax.ShapeDtypeStruct((B, D), jnp.float32),
    grid=(B // BLK,),
    in_specs=[
        pl.BlockSpec((2, D), lambda i: (0, 0)),
        pl.BlockSpec((BLK, D), lambda i: (i, 0)),
    ],
    out_specs=pl.BlockSpec((BLK, D), lambda i: (i, 0)),
)


def kernel(inputs, label, scale_table, offset_table):
    lab = jnp.asarray(label, jnp.int32).reshape(1)
    rows = _lookup_sc(lab, scale_table, offset_table)
    return _affine_tc(rows, inputs)


# speedup vs baseline: 1.1849x; 1.1849x over previous
import functools

import jax
import jax.numpy as jnp
from jax import lax
from jax.experimental import pallas as pl
from jax.experimental.pallas import tpu as pltpu
from jax.experimental.pallas import tpu_sc as plsc

B = 16384
D = 128
NC = 2
NS = 16
NW = NC * NS
LANES = 16
ROWS_PER_W = B // NW
CHUNK_ROWS = 128
CHUNKS = ROWS_PER_W // CHUNK_ROWS
NBUF = 2

_mesh = plsc.VectorSubcoreMesh(core_axis_name="c", subcore_axis_name="s")


@functools.partial(
    pl.kernel,
    mesh=_mesh,
    out_type=jax.ShapeDtypeStruct((B, D), jnp.float32),
    compiler_params=pltpu.CompilerParams(
        use_tc_tiling_on_sc=True, skip_device_barrier=True),
    scratch_types=[
        pltpu.VMEM((1,), jnp.int32),
        pltpu.VMEM((1, D), jnp.float32),
        pltpu.VMEM((1, D), jnp.float32),
        pltpu.VMEM((NBUF, CHUNK_ROWS, D), jnp.float32),
        pltpu.VMEM((NBUF, CHUNK_ROWS, D), jnp.float32),
        pltpu.SemaphoreType.DMA,
    ] + [pltpu.SemaphoreType.DMA] * (2 * NBUF),
)
def _sep_bias_sc(in_hbm, lab_hbm, scale_hbm, off_hbm, out_hbm,
                 idx_v, srow_v, orow_v, ibufs, obufs,
                 gsem, *bufsems):
    wid = lax.axis_index("s") * NC + lax.axis_index("c")
    base = wid * ROWS_PER_W
    si = list(bufsems[:NBUF])
    so = list(bufsems[NBUF:])

    h_in = {}
    for c in range(NBUF):
        h_in[c] = pltpu.async_copy(
            in_hbm.at[pl.ds(base + c * CHUNK_ROWS, CHUNK_ROWS)],
            ibufs.at[c % NBUF], si[c % NBUF])

    pltpu.sync_copy(lab_hbm, idx_v)
    pltpu.async_copy(scale_hbm.at[idx_v], srow_v, gsem).wait()
    pltpu.async_copy(off_hbm.at[idx_v], orow_v, gsem).wait()

    svec = [srow_v[0, pl.ds(LANES * j, LANES)] for j in range(D // LANES)]
    ovec = [orow_v[0, pl.ds(LANES * j, LANES)] for j in range(D // LANES)]

    h_out = {}
    for c in range(CHUNKS):
        b = c % NBUF
        h_in[c].wait()
        if c >= NBUF:
            h_out[c - NBUF].wait()
        ibuf = ibufs.at[b]
        obuf = obufs.at[b]

        def row_body(r, carry, ibuf=ibuf, obuf=obuf):
            for j in range(D // LANES):
                x = ibuf[r, pl.ds(LANES * j, LANES)]
                obuf[r, pl.ds(LANES * j, LANES)] = jnp.maximum(
                    x * svec[j] + ovec[j], 0.0)
            return carry

        lax.fori_loop(0, CHUNK_ROWS, row_body, jnp.int32(0))

        h_out[c] = pltpu.async_copy(
            obuf, out_hbm.at[pl.ds(base + c * CHUNK_ROWS, CHUNK_ROWS)],
            so[b])
        if c + NBUF < CHUNKS:
            h_in[c + NBUF] = pltpu.async_copy(
                in_hbm.at[pl.ds(base + (c + NBUF) * CHUNK_ROWS, CHUNK_ROWS)],
                ibufs.at[b], si[b])

    for c in range(CHUNKS - NBUF, CHUNKS):
        h_out[c].wait()


def kernel(inputs, label, scale_table, offset_table):
    lab = jnp.asarray(label, jnp.int32).reshape(1)
    return _sep_bias_sc(inputs, lab, scale_table, offset_table)


# speedup vs baseline: 1.1929x; 1.0068x over previous
import functools

import jax
import jax.numpy as jnp
from jax import lax
from jax.experimental import pallas as pl
from jax.experimental.pallas import tpu as pltpu
from jax.experimental.pallas import tpu_sc as plsc

B = 16384
D = 128
NC = 2
NS = 16
NW = NC * NS
LANES = 16
ROWS_PER_W = B // NW
CHUNK_ROWS = 128
CHUNKS = ROWS_PER_W // CHUNK_ROWS
NBUF = 2

_mesh = plsc.VectorSubcoreMesh(core_axis_name="c", subcore_axis_name="s")


@functools.partial(
    pl.kernel,
    mesh=_mesh,
    out_type=jax.ShapeDtypeStruct((B, D), jnp.float32),
    compiler_params=pltpu.CompilerParams(use_tc_tiling_on_sc=True),
    scratch_types=[
        pltpu.VMEM((1,), jnp.int32),
        pltpu.VMEM((1, D), jnp.float32),
        pltpu.VMEM((1, D), jnp.float32),
        pltpu.VMEM((NBUF, CHUNK_ROWS, D), jnp.float32),
        pltpu.VMEM((NBUF, CHUNK_ROWS, D), jnp.float32),
        pltpu.SemaphoreType.DMA,
    ] + [pltpu.SemaphoreType.DMA] * (2 * NBUF),
)
def _sep_bias_sc(in_hbm, lab_hbm, scale_hbm, off_hbm, out_hbm,
                 idx_v, srow_v, orow_v, ibufs, obufs,
                 gsem, *bufsems):
    wid = lax.axis_index("s") * NC + lax.axis_index("c")
    base = wid * ROWS_PER_W
    si = list(bufsems[:NBUF])
    so = list(bufsems[NBUF:])

    h_in = {}
    for c in range(NBUF):
        h_in[c] = pltpu.async_copy(
            in_hbm.at[pl.ds(base + c * CHUNK_ROWS, CHUNK_ROWS)],
            ibufs.at[c % NBUF], si[c % NBUF])

    pltpu.sync_copy(lab_hbm, idx_v)
    pltpu.async_copy(scale_hbm.at[idx_v], srow_v, gsem).wait()
    pltpu.async_copy(off_hbm.at[idx_v], orow_v, gsem).wait()

    svec = [srow_v[0, pl.ds(LANES * j, LANES)] for j in range(D // LANES)]
    ovec = [orow_v[0, pl.ds(LANES * j, LANES)] for j in range(D // LANES)]

    h_out = {}
    for c in range(CHUNKS):
        b = c % NBUF
        h_in[c].wait()
        if c >= NBUF:
            h_out[c - NBUF].wait()
        ibuf = ibufs.at[b]
        obuf = obufs.at[b]

        def row_body(r, carry, ibuf=ibuf, obuf=obuf):
            for j in range(D // LANES):
                x = ibuf[r, pl.ds(LANES * j, LANES)]
                obuf[r, pl.ds(LANES * j, LANES)] = jnp.maximum(
                    x * svec[j] + ovec[j], 0.0)
            return carry

        lax.fori_loop(0, CHUNK_ROWS, row_body, jnp.int32(0))

        h_out[c] = pltpu.async_copy(
            obuf, out_hbm.at[pl.ds(base + c * CHUNK_ROWS, CHUNK_ROWS)],
            so[b])
        if c + NBUF < CHUNKS:
            h_in[c + NBUF] = pltpu.async_copy(
                in_hbm.at[pl.ds(base + (c + NBUF) * CHUNK_ROWS, CHUNK_ROWS)],
                ibufs.at[b], si[b])

    for c in range(CHUNKS - NBUF, CHUNKS):
        h_out[c].wait()


def kernel(inputs, label, scale_table, offset_table):
    lab = jnp.asarray(label, jnp.int32).reshape(1)
    return _sep_bias_sc(inputs, lab, scale_table, offset_table)


# speedup vs baseline: 1.1970x; 1.0034x over previous
import functools

import jax
import jax.numpy as jnp
from jax import lax
from jax.experimental import pallas as pl
from jax.experimental.pallas import tpu as pltpu
from jax.experimental.pallas import tpu_sc as plsc

B = 16384
D = 128
NC = 2
NS = 16
NW = NC * NS
LANES = 16
ROWS_PER_W = B // NW
CHUNK_ROWS = 256
CHUNKS = ROWS_PER_W // CHUNK_ROWS
NBUF = 2

_mesh = plsc.VectorSubcoreMesh(core_axis_name="c", subcore_axis_name="s")


@functools.partial(
    pl.kernel,
    mesh=_mesh,
    out_type=jax.ShapeDtypeStruct((B, D), jnp.float32),
    compiler_params=pltpu.CompilerParams(use_tc_tiling_on_sc=True),
    scratch_types=[
        pltpu.VMEM((1,), jnp.int32),
        pltpu.VMEM((1, D), jnp.float32),
        pltpu.VMEM((1, D), jnp.float32),
        pltpu.VMEM((NBUF, CHUNK_ROWS, D), jnp.float32),
        pltpu.SemaphoreType.DMA,
    ] + [pltpu.SemaphoreType.DMA] * (2 * NBUF),
)
def _sep_bias_sc(in_hbm, lab_hbm, scale_hbm, off_hbm, out_hbm,
                 idx_v, srow_v, orow_v, ibufs,
                 gsem, *bufsems):
    wid = lax.axis_index("s") * NC + lax.axis_index("c")
    base = wid * ROWS_PER_W
    si = list(bufsems[:NBUF])
    so = list(bufsems[NBUF:])

    h_in = {}
    for c in range(NBUF):
        h_in[c] = pltpu.async_copy(
            in_hbm.at[pl.ds(base + c * CHUNK_ROWS, CHUNK_ROWS)],
            ibufs.at[c % NBUF], si[c % NBUF])

    pltpu.sync_copy(lab_hbm, idx_v)
    pltpu.async_copy(scale_hbm.at[idx_v], srow_v, gsem).wait()
    pltpu.async_copy(off_hbm.at[idx_v], orow_v, gsem).wait()

    svec = [srow_v[0, pl.ds(LANES * j, LANES)] for j in range(D // LANES)]
    ovec = [orow_v[0, pl.ds(LANES * j, LANES)] for j in range(D // LANES)]

    h_out = {}
    for c in range(CHUNKS):
        b = c % NBUF
        h_in[c].wait()
        if c >= NBUF:
            h_out[c - NBUF].wait()
        ibuf = ibufs.at[b]
        obuf = ibufs.at[b]

        def row_body(r, carry, ibuf=ibuf, obuf=obuf):
            for j in range(D // LANES):
                x = ibuf[r, pl.ds(LANES * j, LANES)]
                obuf[r, pl.ds(LANES * j, LANES)] = jnp.maximum(
                    x * svec[j] + ovec[j], 0.0)
            return carry

        lax.fori_loop(0, CHUNK_ROWS, row_body, jnp.int32(0))

        h_out[c] = pltpu.async_copy(
            obuf, out_hbm.at[pl.ds(base + c * CHUNK_ROWS, CHUNK_ROWS)],
            so[b])
        if c + NBUF < CHUNKS:
            h_in[c + NBUF] = pltpu.async_copy(
                in_hbm.at[pl.ds(base + (c + NBUF) * CHUNK_ROWS, CHUNK_ROWS)],
                ibufs.at[b], si[b])

    for c in range(CHUNKS - NBUF, CHUNKS):
        h_out[c].wait()


def kernel(inputs, label, scale_table, offset_table):
    lab = jnp.asarray(label, jnp.int32).reshape(1)
    return _sep_bias_sc(inputs, lab, scale_table, offset_table)
